# baseline (speedup 1.0000x reference)
import functools

import jax
import jax.numpy as jnp
from jax import lax
from jax.experimental import pallas as pl
from jax.experimental.pallas import tpu as pltpu
from jax.experimental.pallas import tpu_sc as plsc

N = 10000
E = 320000
D = 128
HEADS = 8
DIM_ENC = 128
DIM_MLP = 256
NUM_GRAPHS = 64

LANES = 16
NC = 2
NS = 16
NB = 10
BN = N // NB
N_PAD = 10240
ROWS_PER_TILE = N_PAD // NS

EA_PER_W = E // (NC * NS)
EA_B = 2000
EB_PER_T = E // NS
EB_B = 80
EB_NCH = EB_PER_T // EB_B
HALF = DIM_ENC // 2


def _cdiv(a, b):
    return (a + b - 1) // b


def _tc_attn_body(din, x_ref, w_ref, as_ref, ad_ref, asn_ref, adn_ref):
    w3 = w_ref[...].reshape(din, HEADS, DIM_ENC)
    pad = jnp.zeros((din, LANES - HEADS), jnp.float32)
    was = jnp.concatenate([(w3 * as_ref[...][None]).sum(-1), pad], axis=1)
    wad = jnp.concatenate([(w3 * ad_ref[...][None]).sum(-1), pad], axis=1)
    asn_ref[...] = jnp.dot(x_ref[...], was,
                           preferred_element_type=jnp.float32)
    adn_ref[...] = jnp.dot(x_ref[...], wad,
                           preferred_element_type=jnp.float32)


def _tc_attn(xl, W, att_s, att_d):
    din = xl.shape[1]
    return pl.pallas_call(
        functools.partial(_tc_attn_body, din),
        grid=(NB,),
        in_specs=[
            pl.BlockSpec((BN, din), lambda i: (i, 0)),
            pl.BlockSpec((din, HEADS * DIM_ENC), lambda i: (0, 0)),
            pl.BlockSpec((HEADS, DIM_ENC), lambda i: (0, 0)),
            pl.BlockSpec((HEADS, DIM_ENC), lambda i: (0, 0)),
        ],
        out_specs=[
            pl.BlockSpec((BN, LANES), lambda i: (i, 0)),
            pl.BlockSpec((BN, LANES), lambda i: (i, 0)),
        ],
        out_shape=[
            jax.ShapeDtypeStruct((N, LANES), jnp.float32),
            jax.ShapeDtypeStruct((N, LANES), jnp.float32),
        ],
    )(xl, W, att_s, att_d)


def _tc_encode_body(x_ref, w_ref, ht_ref):
    h = jnp.dot(x_ref[...], w_ref[...], preferred_element_type=jnp.float32)
    h3 = h.reshape(BN, HEADS, DIM_ENC)
    for k in range(HEADS):
        ht_ref[k] = h3[:, k, :]


def _tc_encode(xl, W):
    din = xl.shape[1]
    return pl.pallas_call(
        _tc_encode_body,
        grid=(NB,),
        in_specs=[
            pl.BlockSpec((BN, din), lambda i: (i, 0)),
            pl.BlockSpec((din, HEADS * DIM_ENC), lambda i: (0, 0)),
        ],
        out_specs=pl.BlockSpec((HEADS, BN, DIM_ENC), lambda i: (0, i, 0)),
        out_shape=jax.ShapeDtypeStruct((HEADS, N, DIM_ENC), jnp.float32),
    )(xl, W)


def _sc_edge_attn_body(src_hbm, dst_hbm, asn_hbm, adn_hbm,
                       ex_hbm, den_hbm,
                       sidx_v, didx_v, asr_v, adr_v, zb_v, den_sh):
    cid = lax.axis_index("c")
    sid = lax.axis_index("s")
    wid = sid * NC + cid

    @pl.loop(0, 128)
    def _(i):
        z = jnp.zeros((LANES,), jnp.float32)
        zb_v[i, :] = z

    @pl.loop(0, ROWS_PER_TILE // 128)
    def _(p):
        pltpu.sync_copy(zb_v, den_sh.at[pl.ds(sid * ROWS_PER_TILE + p * 128, 128)])

    plsc.subcore_barrier()

    @pl.loop(0, EA_PER_W // EA_B)
    def _(i):
        base = wid * EA_PER_W + i * EA_B
        pltpu.sync_copy(src_hbm.at[pl.ds(base, EA_B)], sidx_v)
        pltpu.sync_copy(dst_hbm.at[pl.ds(base, EA_B)], didx_v)
        pltpu.sync_copy(asn_hbm.at[sidx_v], asr_v)
        pltpu.sync_copy(adn_hbm.at[didx_v], adr_v)

        @pl.loop(0, EA_B)
        def _(e):
            v = asr_v[e, :] + adr_v[e, :]
            v = jnp.where(v >= 0.0, v, v * 0.2)
            asr_v[e, :] = jnp.exp(v)

        pltpu.sync_copy(asr_v, ex_hbm.at[pl.ds(base, EA_B)])
        pltpu.sync_copy(asr_v, den_sh.at[didx_v], add=True)

    plsc.subcore_barrier()

    pltpu.sync_copy(den_sh.at[pl.ds(sid * ROWS_PER_TILE, ROWS_PER_TILE)],
                    den_hbm.at[cid].at[pl.ds(sid * ROWS_PER_TILE, ROWS_PER_TILE)])


_SC_PARAMS = pltpu.CompilerParams(use_tc_tiling_on_sc=False,
                                  needs_layout_passes=False)


def _sc_edge_attn(src, dst, asn_p, adn_p):
    mesh = plsc.VectorSubcoreMesh(core_axis_name="c", subcore_axis_name="s")
    k = pl.kernel(
        _sc_edge_attn_body,
        mesh=mesh,
        compiler_params=_SC_PARAMS,
        out_type=[
            jax.ShapeDtypeStruct((E, LANES), jnp.float32),
            jax.ShapeDtypeStruct((NC, N_PAD, LANES), jnp.float32),
        ],
        scratch_types=[
            pltpu.VMEM((EA_B,), jnp.int32),
            pltpu.VMEM((EA_B,), jnp.int32),
            pltpu.VMEM((EA_B, LANES), jnp.float32),
            pltpu.VMEM((EA_B, LANES), jnp.float32),
            pltpu.VMEM((128, LANES), jnp.float32),
            pltpu.VMEM_SHARED((N_PAD, LANES), jnp.float32),
        ],
    )
    return k(src, dst, asn_p, adn_p)


def _sc_aggregate_body(s_hbm, d_hbm, ext_hbm, ht_hbm, out_hbm,
                       sbuf_v, exbuf_v, dbuf_v, rows_v, acc_sh,
                       msem, gsem, ssem):
    cid = lax.axis_index("c")
    sid = lax.axis_index("s")

    @pl.loop(0, HEADS // NC)
    def _(rr):
        kk = 2 * rr + cid
        qbase = sid * EB_NCH
        ebase = sid * EB_PER_T

        @pl.loop(0, 80)
        def _(i):
            for j in range(DIM_ENC // LANES):
                rows_v[0, i, pl.ds(j * LANES, LANES)] = jnp.zeros(
                    (LANES,), jnp.float32)

        @pl.loop(0, ROWS_PER_TILE // 80)
        def _(p):
            pltpu.sync_copy(rows_v.at[0].at[pl.ds(0, 80)],
                            acc_sh.at[pl.ds(sid * ROWS_PER_TILE + p * 80,
                                            80)])

        plsc.subcore_barrier()

        def m_issue(s, i):
            pltpu.async_copy(s_hbm.at[qbase + i], sbuf_v.at[s],
                             msem.at[s])
            pltpu.async_copy(d_hbm.at[qbase + i], dbuf_v.at[i % 4],
                             msem.at[s])
            pltpu.async_copy(ext_hbm.at[kk].at[pl.ds(ebase + i * EB_B,
                                                     EB_B)],
                             exbuf_v.at[s], msem.at[s])

        def m_wait(s):
            pltpu.make_async_copy(s_hbm.at[qbase], sbuf_v.at[s],
                                  msem.at[s]).wait()
            pltpu.make_async_copy(d_hbm.at[qbase], dbuf_v.at[0],
                                  msem.at[s]).wait()
            pltpu.make_async_copy(ext_hbm.at[kk].at[pl.ds(0, EB_B)],
                                  exbuf_v.at[s], msem.at[s]).wait()

        def g_issue(s):
            pltpu.async_copy(ht_hbm.at[kk].at[sbuf_v.at[s]],
                             rows_v.at[s], gsem.at[s])

        def g_wait(s):
            pltpu.make_async_copy(ht_hbm.at[kk].at[sbuf_v.at[s]],
                                  rows_v.at[s], gsem.at[s]).wait()

        def s_issue(s, i):
            pltpu.async_copy(rows_v.at[s], acc_sh.at[dbuf_v.at[i % 4]],
                             ssem.at[s], add=True)

        def s_wait(s):
            pltpu.make_async_copy(rows_v.at[s], acc_sh.at[dbuf_v.at[0]],
                                  ssem.at[s]).wait()

        def compute(s):
            exrow = exbuf_v.at[s]

            @plsc.parallel_loop(0, EB_B, step=1, unroll=8)
            def _(e):
                ev = lax.broadcast_in_dim(e, (LANES,), ())
                sv = plsc.load_gather(exrow, [ev])
                for j in range(DIM_ENC // LANES):
                    rows_v[s, e, pl.ds(j * LANES, LANES)] = (
                        rows_v[s, e, pl.ds(j * LANES, LANES)] * sv)

        def process(i, s, do_m_issue, do_g_issue):
            s1, s2 = (s + 1) % 3, (s + 2) % 3
            if do_g_issue:
                m_wait(s1)
                if isinstance(i, int):
                    if i >= 2:
                        s_wait(s1)
                else:
                    @pl.when(i >= 2)
                    def _():
                        s_wait(s1)

                g_issue(s1)
            if do_m_issue:
                m_issue(s2, i + 2)
            g_wait(s)
            compute(s)
            s_issue(s, i)

        m_issue(0, 0)
        m_wait(0)
        g_issue(0)
        m_issue(1, 1)

        rem = (EB_NCH - 2) % 3
        main = EB_NCH - 2 - rem

        @pl.loop(0, main, step=3)
        def _(i):
            process(i, 0, True, True)
            process(i + 1, 1, True, True)
            process(i + 2, 2, True, True)

        for c in range(main, EB_NCH - 2):
            process(c, c % 3, True, True)
        process(EB_NCH - 2, (EB_NCH - 2) % 3, False, True)
        process(EB_NCH - 1, (EB_NCH - 1) % 3, False, False)
        s_wait((EB_NCH - 3) % 3)
        s_wait((EB_NCH - 2) % 3)
        s_wait((EB_NCH - 1) % 3)

        plsc.subcore_barrier()

        pltpu.sync_copy(
            acc_sh.at[pl.ds(sid * ROWS_PER_TILE, ROWS_PER_TILE)],
            out_hbm.at[kk].at[pl.ds(sid * ROWS_PER_TILE, ROWS_PER_TILE)])

        plsc.subcore_barrier()


def _sc_aggregate(srcQ, dstQ, exT, hT):
    mesh = plsc.VectorSubcoreMesh(core_axis_name="c", subcore_axis_name="s")
    k = pl.kernel(
        _sc_aggregate_body,
        mesh=mesh,
        compiler_params=_SC_PARAMS,
        out_type=jax.ShapeDtypeStruct((HEADS, N_PAD, DIM_ENC), jnp.float32),
        scratch_types=[
            pltpu.VMEM((3, EB_B), jnp.int32),
            pltpu.VMEM((3, EB_B), jnp.float32),
            pltpu.VMEM((4, EB_B), jnp.int32),
            pltpu.VMEM((3, EB_B, DIM_ENC), jnp.float32),
            pltpu.VMEM_SHARED((N_PAD, DIM_ENC), jnp.float32),
            pltpu.SemaphoreType.DMA((3,)),
            pltpu.SemaphoreType.DMA((3,)),
            pltpu.SemaphoreType.DMA((3,)),
        ],
    )
    return k(srcQ, dstQ, exT, hT)


def _tc_combine_body(concat, out_ref, ht_ref, den_ref, asn_ref, adn_ref,
                     b_ref, o_ref):
    a = asn_ref[...][:, :HEADS] + adn_ref[...][:, :HEADS]
    a = jnp.where(a >= 0.0, a, a * 0.2)
    exl = jnp.exp(a)
    den = (den_ref[...][0, :, :HEADS] + den_ref[...][1, :, :HEADS]
           + exl + 1e-16)
    b = b_ref[...][0]
    if concat:
        for k in range(HEADS):
            c0 = k * DIM_ENC
            num = out_ref[k] + exl[:, k:k + 1] * ht_ref[k]
            o_ref[:, c0:c0 + DIM_ENC] = (num / den[:, k:k + 1]
                                         + b[c0:c0 + DIM_ENC][None, :])
    else:
        acc = jnp.zeros((BN, DIM_ENC), jnp.float32)
        for k in range(HEADS):
            acc = acc + ((out_ref[k] + exl[:, k:k + 1] * ht_ref[k])
                         / den[:, k:k + 1])
        o_ref[...] = acc * (1.0 / HEADS) + b[None, :]


def _tc_combine(outH, hT, denP, asn_p, adn_p, bias, concat):
    dout = HEADS * DIM_ENC if concat else DIM_ENC
    return pl.pallas_call(
        functools.partial(_tc_combine_body, concat),
        grid=(NB,),
        in_specs=[
            pl.BlockSpec((HEADS, BN, DIM_ENC), lambda i: (0, i, 0)),
            pl.BlockSpec((HEADS, BN, DIM_ENC), lambda i: (0, i, 0)),
            pl.BlockSpec((NC, BN, LANES), lambda i: (0, i, 0)),
            pl.BlockSpec((BN, LANES), lambda i: (i, 0)),
            pl.BlockSpec((BN, LANES), lambda i: (i, 0)),
            pl.BlockSpec((1, dout), lambda i: (0, 0)),
        ],
        out_specs=pl.BlockSpec((BN, dout), lambda i: (i, 0)),
        out_shape=jax.ShapeDtypeStruct((N, dout), jnp.float32),
    )(outH, hT, denP, asn_p, adn_p, bias.reshape(1, dout))


def _readout_mlp_kernel(h_ref, b_ref, fw1_ref, fb1_ref, fw2_ref, fb2_ref,
                        fw3_ref, fb3_ref, o_ref):
    b = b_ref[0, :]
    onehot = (b[None, :] == jax.lax.broadcasted_iota(
        jnp.int32, (NUM_GRAPHS, N), 0)).astype(jnp.float32)
    g = jnp.dot(onehot, h_ref[...], preferred_element_type=jnp.float32)
    g = jnp.maximum(jnp.dot(g, fw1_ref[...],
                            preferred_element_type=jnp.float32)
                    + fb1_ref[0, :][None, :], 0.0)
    g = jnp.maximum(jnp.dot(g, fw2_ref[...],
                            preferred_element_type=jnp.float32)
                    + fb2_ref[0, :][None, :], 0.0)
    o_ref[...] = jnp.dot(g, fw3_ref[...],
                         preferred_element_type=jnp.float32) + fb3_ref[0, :][None, :]


def _gat_layer_fast(xl, srcQ, dstQ, src, dst, W, att_s, att_d, bias, concat):
    asn_p, adn_p = _tc_attn(xl, W, att_s, att_d)
    exE, denP = _sc_edge_attn(src, dst, asn_p, adn_p)
    hT = _tc_encode(xl, W)
    exT = exE[:, :HEADS].T
    outH = _sc_aggregate(srcQ, dstQ, exT, hT)
    return _tc_combine(outH[:, :N, :], hT, denP[:, :N, :], asn_p, adn_p,
                       bias, concat)


def kernel(x, edge_index, batch, W1, a1s, a1d, b1, W2, a2s, a2d, b2,
           fw1, fb1, fw2, fb2, fw3, fb3):
    src = edge_index[0].astype(jnp.int32)
    dst = edge_index[1].astype(jnp.int32)
    srcQ = src.reshape(NS * EB_NCH, EB_B)
    dstQ = dst.reshape(NS * EB_NCH, EB_B)
    o1 = _gat_layer_fast(x, srcQ, dstQ, src, dst, W1, a1s, a1d, b1,
                         concat=False)
    o2 = _gat_layer_fast(o1, srcQ, dstQ, src, dst, W2, a2s, a2d, b2,
                         concat=True)
    out = pl.pallas_call(
        _readout_mlp_kernel,
        out_shape=jax.ShapeDtypeStruct((NUM_GRAPHS, 1), jnp.float32),
    )(o2, batch.reshape(1, N).astype(jnp.int32),
      fw1, fb1.reshape(1, -1), fw2, fb2.reshape(1, -1),
      fw3, fb3.reshape(1, -1))
    return out


# speedup vs baseline: 1.0023x; 1.0023x over previous
import functools

import jax
import jax.numpy as jnp
from jax import lax
from jax.experimental import pallas as pl
from jax.experimental.pallas import tpu as pltpu
from jax.experimental.pallas import tpu_sc as plsc

N = 10000
E = 320000
D = 128
HEADS = 8
DIM_ENC = 128
DIM_MLP = 256
NUM_GRAPHS = 64

LANES = 16
NC = 2
NS = 16
NB = 10
BN = N // NB
N_PAD = 10240
ROWS_PER_TILE = N_PAD // NS

EA_PER_W = E // (NC * NS)
EA_B = 2000
EB_PER_T = E // NS
EB_B = 80
EB_NCH = EB_PER_T // EB_B
HALF = DIM_ENC // 2


def _cdiv(a, b):
    return (a + b - 1) // b


def _tc_attn_body(din, x_ref, w_ref, as_ref, ad_ref, asn_ref, adn_ref):
    w3 = w_ref[...].reshape(din, HEADS, DIM_ENC)
    pad = jnp.zeros((din, LANES - HEADS), jnp.float32)
    was = jnp.concatenate([(w3 * as_ref[...][None]).sum(-1), pad], axis=1)
    wad = jnp.concatenate([(w3 * ad_ref[...][None]).sum(-1), pad], axis=1)
    asn_ref[...] = jnp.dot(x_ref[...], was,
                           preferred_element_type=jnp.float32)
    adn_ref[...] = jnp.dot(x_ref[...], wad,
                           preferred_element_type=jnp.float32)


def _tc_attn(xl, W, att_s, att_d):
    din = xl.shape[1]
    return pl.pallas_call(
        functools.partial(_tc_attn_body, din),
        grid=(NB,),
        in_specs=[
            pl.BlockSpec((BN, din), lambda i: (i, 0)),
            pl.BlockSpec((din, HEADS * DIM_ENC), lambda i: (0, 0)),
            pl.BlockSpec((HEADS, DIM_ENC), lambda i: (0, 0)),
            pl.BlockSpec((HEADS, DIM_ENC), lambda i: (0, 0)),
        ],
        out_specs=[
            pl.BlockSpec((BN, LANES), lambda i: (i, 0)),
            pl.BlockSpec((BN, LANES), lambda i: (i, 0)),
        ],
        out_shape=[
            jax.ShapeDtypeStruct((N, LANES), jnp.float32),
            jax.ShapeDtypeStruct((N, LANES), jnp.float32),
        ],
    )(xl, W, att_s, att_d)


def _tc_encode_body(x_ref, w_ref, ht_ref):
    h = jnp.dot(x_ref[...], w_ref[...], preferred_element_type=jnp.float32)
    h3 = h.reshape(BN, HEADS, DIM_ENC)
    for k in range(HEADS):
        ht_ref[k] = h3[:, k, :]


def _tc_encode(xl, W):
    din = xl.shape[1]
    return pl.pallas_call(
        _tc_encode_body,
        grid=(NB,),
        in_specs=[
            pl.BlockSpec((BN, din), lambda i: (i, 0)),
            pl.BlockSpec((din, HEADS * DIM_ENC), lambda i: (0, 0)),
        ],
        out_specs=pl.BlockSpec((HEADS, BN, DIM_ENC), lambda i: (0, i, 0)),
        out_shape=jax.ShapeDtypeStruct((HEADS, N, DIM_ENC), jnp.float32),
    )(xl, W)


def _sc_edge_attn_body(src_hbm, dst_hbm, asn_hbm, adn_hbm,
                       ex_hbm, den_hbm,
                       sidx_v, didx_v, asr_v, adr_v, zb_v, den_sh):
    cid = lax.axis_index("c")
    sid = lax.axis_index("s")
    wid = sid * NC + cid

    @pl.loop(0, 128)
    def _(i):
        z = jnp.zeros((LANES,), jnp.float32)
        zb_v[i, :] = z

    @pl.loop(0, ROWS_PER_TILE // 128)
    def _(p):
        pltpu.sync_copy(zb_v, den_sh.at[pl.ds(sid * ROWS_PER_TILE + p * 128, 128)])

    plsc.subcore_barrier()

    @pl.loop(0, EA_PER_W // EA_B)
    def _(i):
        base = wid * EA_PER_W + i * EA_B
        pltpu.sync_copy(src_hbm.at[pl.ds(base, EA_B)], sidx_v)
        pltpu.sync_copy(dst_hbm.at[pl.ds(base, EA_B)], didx_v)
        pltpu.sync_copy(asn_hbm.at[sidx_v], asr_v)
        pltpu.sync_copy(adn_hbm.at[didx_v], adr_v)

        @pl.loop(0, EA_B)
        def _(e):
            v = asr_v[e, :] + adr_v[e, :]
            v = jnp.where(v >= 0.0, v, v * 0.2)
            asr_v[e, :] = jnp.exp(v)

        pltpu.sync_copy(asr_v, ex_hbm.at[pl.ds(base, EA_B)])
        pltpu.sync_copy(asr_v, den_sh.at[didx_v], add=True)

    plsc.subcore_barrier()

    pltpu.sync_copy(den_sh.at[pl.ds(sid * ROWS_PER_TILE, ROWS_PER_TILE)],
                    den_hbm.at[cid].at[pl.ds(sid * ROWS_PER_TILE, ROWS_PER_TILE)])


_SC_PARAMS = pltpu.CompilerParams(use_tc_tiling_on_sc=False,
                                  needs_layout_passes=False)


def _sc_edge_attn(src, dst, asn_p, adn_p):
    mesh = plsc.VectorSubcoreMesh(core_axis_name="c", subcore_axis_name="s")
    k = pl.kernel(
        _sc_edge_attn_body,
        mesh=mesh,
        compiler_params=_SC_PARAMS,
        out_type=[
            jax.ShapeDtypeStruct((E, LANES), jnp.float32),
            jax.ShapeDtypeStruct((NC, N_PAD, LANES), jnp.float32),
        ],
        scratch_types=[
            pltpu.VMEM((EA_B,), jnp.int32),
            pltpu.VMEM((EA_B,), jnp.int32),
            pltpu.VMEM((EA_B, LANES), jnp.float32),
            pltpu.VMEM((EA_B, LANES), jnp.float32),
            pltpu.VMEM((128, LANES), jnp.float32),
            pltpu.VMEM_SHARED((N_PAD, LANES), jnp.float32),
        ],
    )
    return k(src, dst, asn_p, adn_p)


def _sc_aggregate_body(s_hbm, d_hbm, ext_hbm, ht_hbm, out_hbm,
                       sbuf_v, exbuf_v, dbuf_v, rows_v, acc_sh,
                       msem, gsem, ssem):
    cid = lax.axis_index("c")
    sid = lax.axis_index("s")

    @pl.loop(0, HEADS // NC)
    def _(rr):
        kk = 2 * rr + cid
        qbase = sid * EB_NCH
        ebase = sid * EB_PER_T

        @pl.loop(0, 80)
        def _(i):
            for j in range(DIM_ENC // LANES):
                rows_v[0, i, pl.ds(j * LANES, LANES)] = jnp.zeros(
                    (LANES,), jnp.float32)

        @pl.loop(0, ROWS_PER_TILE // 80)
        def _(p):
            pltpu.sync_copy(rows_v.at[0].at[pl.ds(0, 80)],
                            acc_sh.at[pl.ds(sid * ROWS_PER_TILE + p * 80,
                                            80)])

        plsc.subcore_barrier()

        def m_issue(s, i):
            pltpu.async_copy(s_hbm.at[qbase + i], sbuf_v.at[s],
                             msem.at[s])
            pltpu.async_copy(d_hbm.at[qbase + i], dbuf_v.at[i % 4],
                             msem.at[s])
            pltpu.async_copy(ext_hbm.at[kk].at[pl.ds(ebase + i * EB_B,
                                                     EB_B)],
                             exbuf_v.at[s], msem.at[s])

        def m_wait(s):
            pltpu.make_async_copy(s_hbm.at[qbase], sbuf_v.at[s],
                                  msem.at[s]).wait()
            pltpu.make_async_copy(d_hbm.at[qbase], dbuf_v.at[0],
                                  msem.at[s]).wait()
            pltpu.make_async_copy(ext_hbm.at[kk].at[pl.ds(0, EB_B)],
                                  exbuf_v.at[s], msem.at[s]).wait()

        def g_issue(s):
            pltpu.async_copy(ht_hbm.at[kk].at[sbuf_v.at[s]],
                             rows_v.at[s], gsem.at[s])

        def g_wait(s):
            pltpu.make_async_copy(ht_hbm.at[kk].at[sbuf_v.at[s]],
                                  rows_v.at[s], gsem.at[s]).wait()

        def s_issue(s, i):
            pltpu.async_copy(rows_v.at[s], acc_sh.at[dbuf_v.at[i % 4]],
                             ssem.at[s], add=True)

        def s_wait(s):
            pltpu.make_async_copy(rows_v.at[s], acc_sh.at[dbuf_v.at[0]],
                                  ssem.at[s]).wait()

        def compute(s):
            exrow = exbuf_v.at[s]

            @plsc.parallel_loop(0, EB_B, step=1, unroll=4)
            def _(e):
                ev = lax.broadcast_in_dim(e, (LANES,), ())
                sv = plsc.load_gather(exrow, [ev])
                for j in range(DIM_ENC // LANES):
                    rows_v[s, e, pl.ds(j * LANES, LANES)] = (
                        rows_v[s, e, pl.ds(j * LANES, LANES)] * sv)

        def process(i, s, do_m_issue, do_g_issue):
            s1, s2 = (s + 1) % 3, (s + 2) % 3
            if do_g_issue:
                m_wait(s1)
                if isinstance(i, int):
                    if i >= 2:
                        s_wait(s1)
                else:
                    @pl.when(i >= 2)
                    def _():
                        s_wait(s1)

                g_issue(s1)
            if do_m_issue:
                m_issue(s2, i + 2)
            g_wait(s)
            compute(s)
            s_issue(s, i)

        m_issue(0, 0)
        m_wait(0)
        g_issue(0)
        m_issue(1, 1)

        rem = (EB_NCH - 2) % 3
        main = EB_NCH - 2 - rem

        @pl.loop(0, main, step=3)
        def _(i):
            process(i, 0, True, True)
            process(i + 1, 1, True, True)
            process(i + 2, 2, True, True)

        for c in range(main, EB_NCH - 2):
            process(c, c % 3, True, True)
        process(EB_NCH - 2, (EB_NCH - 2) % 3, False, True)
        process(EB_NCH - 1, (EB_NCH - 1) % 3, False, False)
        s_wait((EB_NCH - 3) % 3)
        s_wait((EB_NCH - 2) % 3)
        s_wait((EB_NCH - 1) % 3)

        plsc.subcore_barrier()

        pltpu.sync_copy(
            acc_sh.at[pl.ds(sid * ROWS_PER_TILE, ROWS_PER_TILE)],
            out_hbm.at[kk].at[pl.ds(sid * ROWS_PER_TILE, ROWS_PER_TILE)])

        plsc.subcore_barrier()


def _sc_aggregate(srcQ, dstQ, exT, hT):
    mesh = plsc.VectorSubcoreMesh(core_axis_name="c", subcore_axis_name="s")
    k = pl.kernel(
        _sc_aggregate_body,
        mesh=mesh,
        compiler_params=_SC_PARAMS,
        out_type=jax.ShapeDtypeStruct((HEADS, N_PAD, DIM_ENC), jnp.float32),
        scratch_types=[
            pltpu.VMEM((3, EB_B), jnp.int32),
            pltpu.VMEM((3, EB_B), jnp.float32),
            pltpu.VMEM((4, EB_B), jnp.int32),
            pltpu.VMEM((3, EB_B, DIM_ENC), jnp.float32),
            pltpu.VMEM_SHARED((N_PAD, DIM_ENC), jnp.float32),
            pltpu.SemaphoreType.DMA((3,)),
            pltpu.SemaphoreType.DMA((3,)),
            pltpu.SemaphoreType.DMA((3,)),
        ],
    )
    return k(srcQ, dstQ, exT, hT)


def _tc_combine_body(concat, out_ref, ht_ref, den_ref, asn_ref, adn_ref,
                     b_ref, o_ref):
    a = asn_ref[...][:, :HEADS] + adn_ref[...][:, :HEADS]
    a = jnp.where(a >= 0.0, a, a * 0.2)
    exl = jnp.exp(a)
    den = (den_ref[...][0, :, :HEADS] + den_ref[...][1, :, :HEADS]
           + exl + 1e-16)
    b = b_ref[...][0]
    if concat:
        for k in range(HEADS):
            c0 = k * DIM_ENC
            num = out_ref[k] + exl[:, k:k + 1] * ht_ref[k]
            o_ref[:, c0:c0 + DIM_ENC] = (num / den[:, k:k + 1]
                                         + b[c0:c0 + DIM_ENC][None, :])
    else:
        acc = jnp.zeros((BN, DIM_ENC), jnp.float32)
        for k in range(HEADS):
            acc = acc + ((out_ref[k] + exl[:, k:k + 1] * ht_ref[k])
                         / den[:, k:k + 1])
        o_ref[...] = acc * (1.0 / HEADS) + b[None, :]


def _tc_combine(outH, hT, denP, asn_p, adn_p, bias, concat):
    dout = HEADS * DIM_ENC if concat else DIM_ENC
    return pl.pallas_call(
        functools.partial(_tc_combine_body, concat),
        grid=(NB,),
        in_specs=[
            pl.BlockSpec((HEADS, BN, DIM_ENC), lambda i: (0, i, 0)),
            pl.BlockSpec((HEADS, BN, DIM_ENC), lambda i: (0, i, 0)),
            pl.BlockSpec((NC, BN, LANES), lambda i: (0, i, 0)),
            pl.BlockSpec((BN, LANES), lambda i: (i, 0)),
            pl.BlockSpec((BN, LANES), lambda i: (i, 0)),
            pl.BlockSpec((1, dout), lambda i: (0, 0)),
        ],
        out_specs=pl.BlockSpec((BN, dout), lambda i: (i, 0)),
        out_shape=jax.ShapeDtypeStruct((N, dout), jnp.float32),
    )(outH, hT, denP, asn_p, adn_p, bias.reshape(1, dout))


def _readout_mlp_kernel(h_ref, b_ref, fw1_ref, fb1_ref, fw2_ref, fb2_ref,
                        fw3_ref, fb3_ref, o_ref):
    b = b_ref[0, :]
    onehot = (b[None, :] == jax.lax.broadcasted_iota(
        jnp.int32, (NUM_GRAPHS, N), 0)).astype(jnp.float32)
    g = jnp.dot(onehot, h_ref[...], preferred_element_type=jnp.float32)
    g = jnp.maximum(jnp.dot(g, fw1_ref[...],
                            preferred_element_type=jnp.float32)
                    + fb1_ref[0, :][None, :], 0.0)
    g = jnp.maximum(jnp.dot(g, fw2_ref[...],
                            preferred_element_type=jnp.float32)
                    + fb2_ref[0, :][None, :], 0.0)
    o_ref[...] = jnp.dot(g, fw3_ref[...],
                         preferred_element_type=jnp.float32) + fb3_ref[0, :][None, :]


def _gat_layer_fast(xl, srcQ, dstQ, src, dst, W, att_s, att_d, bias, concat):
    asn_p, adn_p = _tc_attn(xl, W, att_s, att_d)
    exE, denP = _sc_edge_attn(src, dst, asn_p, adn_p)
    hT = _tc_encode(xl, W)
    exT = exE[:, :HEADS].T
    outH = _sc_aggregate(srcQ, dstQ, exT, hT)
    return _tc_combine(outH[:, :N, :], hT, denP[:, :N, :], asn_p, adn_p,
                       bias, concat)


def kernel(x, edge_index, batch, W1, a1s, a1d, b1, W2, a2s, a2d, b2,
           fw1, fb1, fw2, fb2, fw3, fb3):
    src = edge_index[0].astype(jnp.int32)
    dst = edge_index[1].astype(jnp.int32)
    srcQ = src.reshape(NS * EB_NCH, EB_B)
    dstQ = dst.reshape(NS * EB_NCH, EB_B)
    o1 = _gat_layer_fast(x, srcQ, dstQ, src, dst, W1, a1s, a1d, b1,
                         concat=False)
    o2 = _gat_layer_fast(o1, srcQ, dstQ, src, dst, W2, a2s, a2d, b2,
                         concat=True)
    out = pl.pallas_call(
        _readout_mlp_kernel,
        out_shape=jax.ShapeDtypeStruct((NUM_GRAPHS, 1), jnp.float32),
    )(o2, batch.reshape(1, N).astype(jnp.int32),
      fw1, fb1.reshape(1, -1), fw2, fb2.reshape(1, -1),
      fw3, fb3.reshape(1, -1))
    return out
